# baseline (speedup 1.0000x reference)
import functools

import jax
import jax.numpy as jnp
from jax import lax
from jax.experimental import pallas as pl
from jax.experimental.pallas import tpu as pltpu
from jax.experimental.pallas import tpu_sc as plsc

NUM = 1000000
DIM = 32
BATCH = 16384
FIELDS = 26
B_TOTAL = BATCH * FIELDS

_info = plsc.get_sparse_core_info()
_NC, _NS = _info.num_cores, _info.num_subcores
_NW = _NC * _NS
_BPW = B_TOTAL // _NW
_CHUNK = 1664
_NCH = _BPW // _CHUNK

_LANES = 16
_JB = 2
_FULL_TILES = (NUM // 128)
_REM = NUM - _FULL_TILES * 128
_NGROUPS = _FULL_TILES // _JB

_mesh = plsc.VectorSubcoreMesh(core_axis_name="c", subcore_axis_name="s")


def _iota16(off=0):
    v = lax.broadcasted_iota(jnp.int32, (_LANES,), 0)
    return v + off if off else v


def _transpose_block(in_v, out_v, nj):
    row0 = _iota16(0)
    row1 = _iota16(_LANES)

    def rbody(r, _):
        for u in range(nj):
            for q in range(4):
                col = jnp.full((_LANES,), 128 * u + q, jnp.int32) + 4 * r
                g0 = plsc.load_gather(in_v, [row0, col])
                g1 = plsc.load_gather(in_v, [row1, col])
                out_v[32 * u + r, pl.ds(32 * q, _LANES)] = g0
                out_v[32 * u + r, pl.ds(32 * q + _LANES, _LANES)] = g1
        return ()

    lax.fori_loop(0, 32, rbody, (), unroll=False)


@functools.partial(
    pl.kernel,
    mesh=_mesh,
    out_type=jax.ShapeDtypeStruct((NUM * DIM // 128, 128), jnp.float32),
    scratch_types=[
        pltpu.VMEM((DIM, 128 * _JB), jnp.float32),
        pltpu.VMEM((DIM, 128 * _JB), jnp.float32),
        pltpu.VMEM((32 * _JB, 128), jnp.float32),
        pltpu.VMEM((32 * _JB, 128), jnp.float32),
        pltpu.VMEM((_REM, DIM), jnp.float32),
        pltpu.SemaphoreType.DMA,
        pltpu.SemaphoreType.DMA,
        pltpu.SemaphoreType.DMA,
        pltpu.SemaphoreType.DMA,
    ],
    compiler_params=pltpu.CompilerParams(needs_layout_passes=False),
)
def _sc_transpose(table_t_hbm, tail_hbm, out_hbm, in0, in1, ob0, ob1, tv,
                  si0, si1, so0, so1):
    wid = lax.axis_index("s") * _NC + lax.axis_index("c")
    nbase = _NGROUPS // _NW
    extra = _NGROUPS - nbase * _NW
    cnt = jnp.where(wid < extra, nbase + 1, nbase)
    g0 = wid * nbase + jnp.minimum(wid, extra)

    ins, obs = [in0, in1], [ob0, ob1]
    sis, sos = [si0, si1], [so0, so1]

    def in_copy(g, b):
        return pltpu.make_async_copy(
            table_t_hbm.at[:, pl.ds(g * (128 * _JB), 128 * _JB)],
            ins[b], sis[b])

    def out_copy(g, b):
        return pltpu.make_async_copy(
            obs[b], out_hbm.at[pl.ds(g * (32 * _JB), 32 * _JB)], sos[b])

    in_copy(g0, 0).start()
    in_copy(g0 + 1, 1).start()

    def pair_body(k, _):
        for b in (0, 1):
            i = 2 * k + b
            g = g0 + i
            in_copy(g, b).wait()

            @pl.when(i >= 2)
            def _():
                out_copy(g - 2, b).wait()

            _transpose_block(ins[b], obs[b], _JB)
            out_copy(g, b).start()

            @pl.when(i + 2 < cnt)
            def _():
                in_copy(g + 2, b).start()
        return ()

    lax.fori_loop(0, cnt // 2, pair_body, (), unroll=False)

    @pl.when(lax.rem(cnt, 2) == 1)
    def _():
        i = cnt - 1
        g = g0 + i
        in_copy(g, 0).wait()
        out_copy(g - 2, 0).wait()
        _transpose_block(ins[0], obs[0], _JB)
        out_copy(g, 0).start()

    out_copy(g0, 0).wait()
    out_copy(g0, 1).wait()

    @pl.when(wid == _NW - 1)
    def _():
        pltpu.sync_copy(tail_hbm, tv)
        for r in range(_REM // 4):
            for q in range(4):
                ob0[r, pl.ds(32 * q, _LANES)] = \
                    tv[4 * r + q, pl.ds(0, _LANES)]
                ob0[r, pl.ds(32 * q + _LANES, _LANES)] = \
                    tv[4 * r + q, pl.ds(_LANES, _LANES)]
        pltpu.sync_copy(ob0.at[pl.ds(0, _REM // 4)],
                        out_hbm.at[pl.ds(_FULL_TILES * 32, _REM // 4)])


@functools.partial(
    pl.kernel,
    mesh=_mesh,
    out_type=jax.ShapeDtypeStruct((B_TOTAL, DIM), jnp.float32),
    scratch_types=[
        pltpu.VMEM((_CHUNK,), jnp.int32),
        pltpu.VMEM((_CHUNK,), jnp.int32),
        pltpu.VMEM((_CHUNK, DIM), jnp.float32),
        pltpu.VMEM((_CHUNK, DIM), jnp.float32),
        pltpu.SemaphoreType.DMA,
        pltpu.SemaphoreType.DMA,
        pltpu.SemaphoreType.DMA,
        pltpu.SemaphoreType.DMA,
        pltpu.SemaphoreType.DMA,
        pltpu.SemaphoreType.DMA,
    ],
    compiler_params=pltpu.CompilerParams(use_tc_tiling_on_sc=False),
)
def _sc_gather(idx_hbm, table_hbm, out_hbm, ib0, ib1, rb0, rb1,
               si0, si1, sg0, sg1, so0, so1):
    wid = lax.axis_index("s") * _NC + lax.axis_index("c")
    base = wid * _BPW
    ib, rb = [ib0, ib1], [rb0, rb1]
    si, sg, so = [si0, si1], [sg0, sg1], [so0, so1]
    idx_cp = [None] * _NCH
    g_cp = [None] * _NCH
    o_cp = [None] * _NCH

    def start_idx(c):
        idx_cp[c] = pltpu.async_copy(
            idx_hbm.at[pl.ds(base + c * _CHUNK, _CHUNK)], ib[c % 2], si[c % 2])

    start_idx(0)
    start_idx(1)
    for c in range(_NCH):
        b = c % 2
        idx_cp[c].wait()
        if c >= 2:
            o_cp[c - 2].wait()
        g_cp[c] = pltpu.async_copy(table_hbm.at[ib[b]], rb[b], sg[b])
        if c >= 1:
            pb = (c - 1) % 2
            g_cp[c - 1].wait()
            o_cp[c - 1] = pltpu.async_copy(
                rb[pb], out_hbm.at[pl.ds(base + (c - 1) * _CHUNK, _CHUNK)],
                so[pb])
            if c + 1 < _NCH:
                start_idx(c + 1)
    lb = (_NCH - 1) % 2
    g_cp[_NCH - 1].wait()
    o_cp[_NCH - 1] = pltpu.async_copy(
        rb[lb], out_hbm.at[pl.ds(base + (_NCH - 1) * _CHUNK, _CHUNK)], so[lb])
    o_cp[_NCH - 2].wait()
    o_cp[_NCH - 1].wait()


def kernel(indexes, embedding_weight):
    idx_flat = indexes.reshape(-1).astype(jnp.int32)
    tail = embedding_weight[_FULL_TILES * 128:]
    tbl_rm128 = _sc_transpose(embedding_weight.T, tail)
    tbl_rm = tbl_rm128.reshape(NUM, DIM)
    out = _sc_gather(idx_flat, tbl_rm)
    return out.reshape(BATCH, FIELDS, DIM)


# speedup vs baseline: 1.6996x; 1.6996x over previous
import functools

import jax
import jax.numpy as jnp
from jax import lax
from jax.experimental import pallas as pl
from jax.experimental.pallas import tpu as pltpu
from jax.experimental.pallas import tpu_sc as plsc

NUM = 1000000
DIM = 32
BATCH = 16384
FIELDS = 26
B_TOTAL = BATCH * FIELDS

_info = plsc.get_sparse_core_info()
_NC, _NS = _info.num_cores, _info.num_subcores
_NW = _NC * _NS
_BPW = B_TOTAL // _NW
_CHUNK = 1664
_NCH = _BPW // _CHUNK

_LANES = 16
_JB = 2
_FULL_TILES = (NUM // 128)
_REM = NUM - _FULL_TILES * 128
_NGROUPS = _FULL_TILES // _JB

_mesh = plsc.VectorSubcoreMesh(core_axis_name="c", subcore_axis_name="s")


def _iota16(off=0):
    v = lax.broadcasted_iota(jnp.int32, (_LANES,), 0)
    return v + off if off else v


def _transpose_block(in_v, out_v, wrap_ref, outc_ref, nj):
    row0 = _iota16(0)
    row1 = _iota16(_LANES)

    def tbody(t, _):
        wrapc = wrap_ref[t, :]
        outc0 = outc_ref[t, :]
        for u in range(nj):
            for k in range(8):
                x0 = 128 * u + 16 * k
                colv = wrapc + x0
                outr = lax.shift_right_logical(colv, 2)
                g0 = plsc.load_gather(in_v, [row0, colv])
                plsc.store_scatter(out_v, [outr, outc0 + row0], g0)
                g1 = plsc.load_gather(in_v, [row1, colv])
                plsc.store_scatter(out_v, [outr, outc0 + row1], g1)
        return ()

    lax.fori_loop(0, _LANES, tbody, (), unroll=False)


@functools.partial(
    pl.kernel,
    mesh=_mesh,
    out_type=jax.ShapeDtypeStruct((NUM * DIM // 128, 128), jnp.float32),
    scratch_types=[
        pltpu.VMEM((DIM, 128 * _JB), jnp.float32),
        pltpu.VMEM((DIM, 128 * _JB), jnp.float32),
        pltpu.VMEM((32 * _JB, 128), jnp.float32),
        pltpu.VMEM((32 * _JB, 128), jnp.float32),
        pltpu.VMEM((_REM, DIM), jnp.float32),
        pltpu.VMEM((_LANES, _LANES), jnp.int32),
        pltpu.VMEM((_LANES, _LANES), jnp.int32),
        pltpu.SemaphoreType.DMA,
        pltpu.SemaphoreType.DMA,
        pltpu.SemaphoreType.DMA,
        pltpu.SemaphoreType.DMA,
    ],
    compiler_params=pltpu.CompilerParams(needs_layout_passes=False),
)
def _sc_transpose(table_t_hbm, tail_hbm, out_hbm, in0, in1, ob0, ob1, tv,
                  wrap_ref, outc_ref, si0, si1, so0, so1):
    for t in range(_LANES):
        w = jnp.bitwise_and(_iota16(t), _LANES - 1)
        wrap_ref[t, :] = w
        outc_ref[t, :] = lax.shift_left(jnp.bitwise_and(w, 3), 5)
    wid = lax.axis_index("s") * _NC + lax.axis_index("c")
    nbase = _NGROUPS // _NW
    extra = _NGROUPS - nbase * _NW
    cnt = jnp.where(wid < extra, nbase + 1, nbase)
    g0 = wid * nbase + jnp.minimum(wid, extra)

    ins, obs = [in0, in1], [ob0, ob1]
    sis, sos = [si0, si1], [so0, so1]

    def in_copy(g, b):
        return pltpu.make_async_copy(
            table_t_hbm.at[:, pl.ds(g * (128 * _JB), 128 * _JB)],
            ins[b], sis[b])

    def out_copy(g, b):
        return pltpu.make_async_copy(
            obs[b], out_hbm.at[pl.ds(g * (32 * _JB), 32 * _JB)], sos[b])

    in_copy(g0, 0).start()
    in_copy(g0 + 1, 1).start()

    def pair_body(k, _):
        for b in (0, 1):
            i = 2 * k + b
            g = g0 + i
            in_copy(g, b).wait()

            @pl.when(i >= 2)
            def _():
                out_copy(g - 2, b).wait()

            _transpose_block(ins[b], obs[b], wrap_ref, outc_ref, _JB)
            out_copy(g, b).start()

            @pl.when(i + 2 < cnt)
            def _():
                in_copy(g + 2, b).start()
        return ()

    lax.fori_loop(0, cnt // 2, pair_body, (), unroll=False)

    @pl.when(lax.rem(cnt, 2) == 1)
    def _():
        i = cnt - 1
        g = g0 + i
        in_copy(g, 0).wait()
        out_copy(g - 2, 0).wait()
        _transpose_block(ins[0], obs[0], wrap_ref, outc_ref, _JB)
        out_copy(g, 0).start()

    out_copy(g0, 0).wait()
    out_copy(g0, 1).wait()

    @pl.when(wid == _NW - 1)
    def _():
        pltpu.sync_copy(tail_hbm, tv)
        for r in range(_REM // 4):
            for q in range(4):
                ob0[r, pl.ds(32 * q, _LANES)] = \
                    tv[4 * r + q, pl.ds(0, _LANES)]
                ob0[r, pl.ds(32 * q + _LANES, _LANES)] = \
                    tv[4 * r + q, pl.ds(_LANES, _LANES)]
        pltpu.sync_copy(ob0.at[pl.ds(0, _REM // 4)],
                        out_hbm.at[pl.ds(_FULL_TILES * 32, _REM // 4)])


@functools.partial(
    pl.kernel,
    mesh=_mesh,
    out_type=jax.ShapeDtypeStruct((B_TOTAL, DIM), jnp.float32),
    scratch_types=[
        pltpu.VMEM((_CHUNK,), jnp.int32),
        pltpu.VMEM((_CHUNK,), jnp.int32),
        pltpu.VMEM((_CHUNK, DIM), jnp.float32),
        pltpu.VMEM((_CHUNK, DIM), jnp.float32),
        pltpu.SemaphoreType.DMA,
        pltpu.SemaphoreType.DMA,
        pltpu.SemaphoreType.DMA,
        pltpu.SemaphoreType.DMA,
        pltpu.SemaphoreType.DMA,
        pltpu.SemaphoreType.DMA,
    ],
    compiler_params=pltpu.CompilerParams(use_tc_tiling_on_sc=False),
)
def _sc_gather(idx_hbm, table_hbm, out_hbm, ib0, ib1, rb0, rb1,
               si0, si1, sg0, sg1, so0, so1):
    wid = lax.axis_index("s") * _NC + lax.axis_index("c")
    base = wid * _BPW
    ib, rb = [ib0, ib1], [rb0, rb1]
    si, sg, so = [si0, si1], [sg0, sg1], [so0, so1]
    idx_cp = [None] * _NCH
    g_cp = [None] * _NCH
    o_cp = [None] * _NCH

    def start_idx(c):
        idx_cp[c] = pltpu.async_copy(
            idx_hbm.at[pl.ds(base + c * _CHUNK, _CHUNK)], ib[c % 2], si[c % 2])

    start_idx(0)
    start_idx(1)
    for c in range(_NCH):
        b = c % 2
        idx_cp[c].wait()
        if c >= 2:
            o_cp[c - 2].wait()
        g_cp[c] = pltpu.async_copy(table_hbm.at[ib[b]], rb[b], sg[b])
        if c >= 1:
            pb = (c - 1) % 2
            g_cp[c - 1].wait()
            o_cp[c - 1] = pltpu.async_copy(
                rb[pb], out_hbm.at[pl.ds(base + (c - 1) * _CHUNK, _CHUNK)],
                so[pb])
            if c + 1 < _NCH:
                start_idx(c + 1)
    lb = (_NCH - 1) % 2
    g_cp[_NCH - 1].wait()
    o_cp[_NCH - 1] = pltpu.async_copy(
        rb[lb], out_hbm.at[pl.ds(base + (_NCH - 1) * _CHUNK, _CHUNK)], so[lb])
    o_cp[_NCH - 2].wait()
    o_cp[_NCH - 1].wait()


def kernel(indexes, embedding_weight):
    idx_flat = indexes.reshape(-1).astype(jnp.int32)
    tail = embedding_weight[_FULL_TILES * 128:]
    tbl_rm128 = _sc_transpose(embedding_weight.T, tail)
    tbl_rm = tbl_rm128.reshape(NUM, DIM)
    out = _sc_gather(idx_flat, tbl_rm)
    return out.reshape(BATCH, FIELDS, DIM)


# speedup vs baseline: 1.8132x; 1.0668x over previous
import functools

import jax
import jax.numpy as jnp
from jax import lax
from jax.experimental import pallas as pl
from jax.experimental.pallas import tpu as pltpu
from jax.experimental.pallas import tpu_sc as plsc

NUM = 1000000
DIM = 32
BATCH = 16384
FIELDS = 26
B_TOTAL = BATCH * FIELDS

_info = plsc.get_sparse_core_info()
_NC, _NS = _info.num_cores, _info.num_subcores
_NW = _NC * _NS
_BPW = B_TOTAL // _NW
_CHUNK = 1664
_NCH = _BPW // _CHUNK

_LANES = 16
_JB = 4
_FULL_TILES = (NUM // 128)
_REM = NUM - _FULL_TILES * 128
_NGROUPS = _FULL_TILES // _JB

_mesh = plsc.VectorSubcoreMesh(core_axis_name="c", subcore_axis_name="s")


def _iota16(off=0):
    v = lax.broadcasted_iota(jnp.int32, (_LANES,), 0)
    return v + off if off else v


def _transpose_block(in_v, out_v, wrap_ref, outc_ref, nj):
    row0 = _iota16(0)
    row1 = _iota16(_LANES)

    def tbody(t, _):
        wrapc = wrap_ref[t, :]
        outc0 = outc_ref[t, :]
        for u in range(nj):
            for k in range(8):
                x0 = 128 * u + 16 * k
                colv = wrapc + x0
                outr = lax.shift_right_logical(colv, 2)
                g0 = plsc.load_gather(in_v, [row0, colv])
                plsc.store_scatter(out_v, [outr, outc0 + row0], g0)
                g1 = plsc.load_gather(in_v, [row1, colv])
                plsc.store_scatter(out_v, [outr, outc0 + row1], g1)
        return ()

    lax.fori_loop(0, _LANES, tbody, (), unroll=False)


@functools.partial(
    pl.kernel,
    mesh=_mesh,
    out_type=jax.ShapeDtypeStruct((NUM * DIM // 128, 128), jnp.float32),
    scratch_types=[
        pltpu.VMEM((DIM, 128 * _JB), jnp.float32),
        pltpu.VMEM((DIM, 128 * _JB), jnp.float32),
        pltpu.VMEM((32 * _JB, 128), jnp.float32),
        pltpu.VMEM((32 * _JB, 128), jnp.float32),
        pltpu.VMEM((_REM, DIM), jnp.float32),
        pltpu.VMEM((_LANES, _LANES), jnp.int32),
        pltpu.VMEM((_LANES, _LANES), jnp.int32),
        pltpu.SemaphoreType.DMA,
        pltpu.SemaphoreType.DMA,
        pltpu.SemaphoreType.DMA,
        pltpu.SemaphoreType.DMA,
    ],
    compiler_params=pltpu.CompilerParams(needs_layout_passes=False),
)
def _sc_transpose(table_t_hbm, tail_hbm, out_hbm, in0, in1, ob0, ob1, tv,
                  wrap_ref, outc_ref, si0, si1, so0, so1):
    for t in range(_LANES):
        w = jnp.bitwise_and(_iota16(t), _LANES - 1)
        wrap_ref[t, :] = w
        outc_ref[t, :] = lax.shift_left(jnp.bitwise_and(w, 3), 5)
    wid = lax.axis_index("s") * _NC + lax.axis_index("c")
    nbase = _NGROUPS // _NW
    extra = _NGROUPS - nbase * _NW
    cnt = jnp.where(wid < extra, nbase + 1, nbase)
    g0 = wid * nbase + jnp.minimum(wid, extra)

    ins, obs = [in0, in1], [ob0, ob1]
    sis, sos = [si0, si1], [so0, so1]

    def in_copy(g, b):
        return pltpu.make_async_copy(
            table_t_hbm.at[:, pl.ds(g * (128 * _JB), 128 * _JB)],
            ins[b], sis[b])

    def out_copy(g, b):
        return pltpu.make_async_copy(
            obs[b], out_hbm.at[pl.ds(g * (32 * _JB), 32 * _JB)], sos[b])

    in_copy(g0, 0).start()
    in_copy(g0 + 1, 1).start()

    def pair_body(k, _):
        for b in (0, 1):
            i = 2 * k + b
            g = g0 + i
            in_copy(g, b).wait()

            @pl.when(i >= 2)
            def _():
                out_copy(g - 2, b).wait()

            _transpose_block(ins[b], obs[b], wrap_ref, outc_ref, _JB)
            out_copy(g, b).start()

            @pl.when(i + 2 < cnt)
            def _():
                in_copy(g + 2, b).start()
        return ()

    lax.fori_loop(0, cnt // 2, pair_body, (), unroll=False)

    @pl.when(lax.rem(cnt, 2) == 1)
    def _():
        i = cnt - 1
        g = g0 + i
        in_copy(g, 0).wait()
        out_copy(g - 2, 0).wait()
        _transpose_block(ins[0], obs[0], wrap_ref, outc_ref, _JB)
        out_copy(g, 0).start()

    out_copy(g0, 0).wait()
    out_copy(g0, 1).wait()

    @pl.when(wid == _NW - 1)
    def _():
        pltpu.sync_copy(tail_hbm, tv)
        for r in range(_REM // 4):
            for q in range(4):
                ob0[r, pl.ds(32 * q, _LANES)] = \
                    tv[4 * r + q, pl.ds(0, _LANES)]
                ob0[r, pl.ds(32 * q + _LANES, _LANES)] = \
                    tv[4 * r + q, pl.ds(_LANES, _LANES)]
        pltpu.sync_copy(ob0.at[pl.ds(0, _REM // 4)],
                        out_hbm.at[pl.ds(_FULL_TILES * 32, _REM // 4)])


@functools.partial(
    pl.kernel,
    mesh=_mesh,
    out_type=jax.ShapeDtypeStruct((B_TOTAL, DIM), jnp.float32),
    scratch_types=[
        pltpu.VMEM((_CHUNK,), jnp.int32),
        pltpu.VMEM((_CHUNK,), jnp.int32),
        pltpu.VMEM((_CHUNK, DIM), jnp.float32),
        pltpu.VMEM((_CHUNK, DIM), jnp.float32),
        pltpu.SemaphoreType.DMA,
        pltpu.SemaphoreType.DMA,
        pltpu.SemaphoreType.DMA,
        pltpu.SemaphoreType.DMA,
        pltpu.SemaphoreType.DMA,
        pltpu.SemaphoreType.DMA,
    ],
    compiler_params=pltpu.CompilerParams(use_tc_tiling_on_sc=False),
)
def _sc_gather(idx_hbm, table_hbm, out_hbm, ib0, ib1, rb0, rb1,
               si0, si1, sg0, sg1, so0, so1):
    wid = lax.axis_index("s") * _NC + lax.axis_index("c")
    base = wid * _BPW
    ib, rb = [ib0, ib1], [rb0, rb1]
    si, sg, so = [si0, si1], [sg0, sg1], [so0, so1]
    idx_cp = [None] * _NCH
    g_cp = [None] * _NCH
    o_cp = [None] * _NCH

    def start_idx(c):
        idx_cp[c] = pltpu.async_copy(
            idx_hbm.at[pl.ds(base + c * _CHUNK, _CHUNK)], ib[c % 2], si[c % 2])

    start_idx(0)
    start_idx(1)
    for c in range(_NCH):
        b = c % 2
        idx_cp[c].wait()
        if c >= 2:
            o_cp[c - 2].wait()
        g_cp[c] = pltpu.async_copy(table_hbm.at[ib[b]], rb[b], sg[b])
        if c >= 1:
            pb = (c - 1) % 2
            g_cp[c - 1].wait()
            o_cp[c - 1] = pltpu.async_copy(
                rb[pb], out_hbm.at[pl.ds(base + (c - 1) * _CHUNK, _CHUNK)],
                so[pb])
            if c + 1 < _NCH:
                start_idx(c + 1)
    lb = (_NCH - 1) % 2
    g_cp[_NCH - 1].wait()
    o_cp[_NCH - 1] = pltpu.async_copy(
        rb[lb], out_hbm.at[pl.ds(base + (_NCH - 1) * _CHUNK, _CHUNK)], so[lb])
    o_cp[_NCH - 2].wait()
    o_cp[_NCH - 1].wait()


def kernel(indexes, embedding_weight):
    idx_flat = indexes.reshape(-1).astype(jnp.int32)
    tail = embedding_weight[_FULL_TILES * 128:]
    tbl_rm128 = _sc_transpose(embedding_weight.T, tail)
    tbl_rm = tbl_rm128.reshape(NUM, DIM)
    out = _sc_gather(idx_flat, tbl_rm)
    return out.reshape(BATCH, FIELDS, DIM)


# speedup vs baseline: 2.0075x; 1.1072x over previous
import functools

import jax
import jax.numpy as jnp
from jax import lax
from jax.experimental import pallas as pl
from jax.experimental.pallas import tpu as pltpu
from jax.experimental.pallas import tpu_sc as plsc

NUM = 1000000
DIM = 32
BATCH = 16384
FIELDS = 26
B_TOTAL = BATCH * FIELDS

_info = plsc.get_sparse_core_info()
_NC, _NS = _info.num_cores, _info.num_subcores
_NW = _NC * _NS
_BPW = B_TOTAL // _NW
_CHUNK = 1664
_NCH = _BPW // _CHUNK

_LANES = 16
_JB = 4
_FULL_TILES = (NUM // 128)
_REM = NUM - _FULL_TILES * 128
_NGROUPS = _FULL_TILES // _JB

_mesh = plsc.VectorSubcoreMesh(core_axis_name="c", subcore_axis_name="s")


def _iota16(off=0):
    v = lax.broadcasted_iota(jnp.int32, (_LANES,), 0)
    return v + off if off else v


def _transpose_block(in_v, out_v, wrap_ref, outc_ref, nj):
    row0 = _iota16(0)
    row1 = _iota16(_LANES)

    def tbody(t, _):
        wrapc = wrap_ref[t, :]
        outc0 = outc_ref[t, :]
        for u in range(nj):
            for k in range(8):
                x0 = 128 * u + 16 * k
                colv = wrapc + x0
                outr = lax.shift_right_logical(colv, 2)
                g0 = plsc.load_gather(in_v, [row0, colv])
                plsc.store_scatter(out_v, [outr, outc0 + row0], g0)
                g1 = plsc.load_gather(in_v, [row1, colv])
                plsc.store_scatter(out_v, [outr, outc0 + row1], g1)
        return ()

    lax.fori_loop(0, _LANES, tbody, (), unroll=False)


@functools.partial(
    pl.kernel,
    mesh=_mesh,
    out_type=jax.ShapeDtypeStruct((NUM * DIM // 128, 128), jnp.float32),
    scratch_types=[
        pltpu.VMEM((DIM, 128 * _JB), jnp.float32),
        pltpu.VMEM((DIM, 128 * _JB), jnp.float32),
        pltpu.VMEM((32 * _JB, 128), jnp.float32),
        pltpu.VMEM((32 * _JB, 128), jnp.float32),
        pltpu.VMEM((_REM, DIM), jnp.float32),
        pltpu.VMEM((_LANES, _LANES), jnp.int32),
        pltpu.VMEM((_LANES, _LANES), jnp.int32),
        pltpu.SemaphoreType.DMA,
        pltpu.SemaphoreType.DMA,
        pltpu.SemaphoreType.DMA,
        pltpu.SemaphoreType.DMA,
    ],
    compiler_params=pltpu.CompilerParams(needs_layout_passes=False),
)
def _sc_transpose(table_t_hbm, tail_hbm, out_hbm, in0, in1, ob0, ob1, tv,
                  wrap_ref, outc_ref, si0, si1, so0, so1):
    for t in range(_LANES):
        w = jnp.bitwise_and(_iota16(t), _LANES - 1)
        wrap_ref[t, :] = w
        outc_ref[t, :] = lax.shift_left(jnp.bitwise_and(w, 3), 5)
    wid = lax.axis_index("s") * _NC + lax.axis_index("c")
    nbase = _NGROUPS // _NW
    extra = _NGROUPS - nbase * _NW
    cnt = jnp.where(wid < extra, nbase + 1, nbase)
    g0 = wid * nbase + jnp.minimum(wid, extra)

    ins, obs = [in0, in1], [ob0, ob1]
    sis, sos = [si0, si1], [so0, so1]

    def in_copy(g, b):
        return pltpu.make_async_copy(
            table_t_hbm.at[:, pl.ds(g * (128 * _JB), 128 * _JB)],
            ins[b], sis[b])

    def out_copy(g, b):
        return pltpu.make_async_copy(
            obs[b], out_hbm.at[pl.ds(g * (32 * _JB), 32 * _JB)], sos[b])

    in_copy(g0, 0).start()
    in_copy(g0 + 1, 1).start()

    def pair_body(k, _):
        for b in (0, 1):
            i = 2 * k + b
            g = g0 + i
            in_copy(g, b).wait()

            @pl.when(i >= 2)
            def _():
                out_copy(g - 2, b).wait()

            _transpose_block(ins[b], obs[b], wrap_ref, outc_ref, _JB)
            out_copy(g, b).start()

            @pl.when(i + 2 < cnt)
            def _():
                in_copy(g + 2, b).start()
        return ()

    lax.fori_loop(0, cnt // 2, pair_body, (), unroll=False)

    @pl.when(lax.rem(cnt, 2) == 1)
    def _():
        i = cnt - 1
        g = g0 + i
        in_copy(g, 0).wait()
        out_copy(g - 2, 0).wait()
        _transpose_block(ins[0], obs[0], wrap_ref, outc_ref, _JB)
        out_copy(g, 0).start()

    out_copy(g0, 0).wait()
    out_copy(g0, 1).wait()

    @pl.when(wid == _NW - 1)
    def _():
        pltpu.sync_copy(tail_hbm, tv)
        for r in range(_REM // 4):
            for q in range(4):
                ob0[r, pl.ds(32 * q, _LANES)] = \
                    tv[4 * r + q, pl.ds(0, _LANES)]
                ob0[r, pl.ds(32 * q + _LANES, _LANES)] = \
                    tv[4 * r + q, pl.ds(_LANES, _LANES)]
        pltpu.sync_copy(ob0.at[pl.ds(0, _REM // 4)],
                        out_hbm.at[pl.ds(_FULL_TILES * 32, _REM // 4)])


_CK = 256
_NCHT = _BPW // _CK
_CPF = BATCH // _CK


@functools.partial(
    pl.kernel,
    mesh=_mesh,
    out_type=jax.ShapeDtypeStruct((FIELDS, DIM, BATCH), jnp.float32),
    scratch_types=[
        pltpu.VMEM((_CK,), jnp.int32),
        pltpu.VMEM((_CK,), jnp.int32),
        pltpu.VMEM((_CK,), jnp.int32),
        pltpu.VMEM((_CK,), jnp.int32),
        pltpu.VMEM((_CK,), jnp.int32),
        pltpu.VMEM((_CK,), jnp.int32),
        pltpu.VMEM((_CK, 128), jnp.float32),
        pltpu.VMEM((_CK, 128), jnp.float32),
        pltpu.VMEM((DIM, _CK), jnp.float32),
        pltpu.VMEM((DIM, _CK), jnp.float32),
        pltpu.VMEM((_LANES, _LANES), jnp.int32),
        pltpu.SemaphoreType.DMA,
        pltpu.SemaphoreType.DMA,
        pltpu.SemaphoreType.DMA,
        pltpu.SemaphoreType.DMA,
        pltpu.SemaphoreType.DMA,
        pltpu.SemaphoreType.DMA,
    ],
    compiler_params=pltpu.CompilerParams(needs_layout_passes=False),
)
def _sc_gather_t(idx_hbm, table_hbm, out_hbm,
                 ib0, ib1, i40, i41, pc0, pc1, rb0, rb1, ob0, ob1,
                 wrap_ref, si0, si1, sg0, sg1, so0, so1):
    for t in range(_LANES):
        wrap_ref[t, :] = jnp.bitwise_and(_iota16(t), _LANES - 1)

    wid = lax.axis_index("s") * _NC + lax.axis_index("c")
    gbase = wid * _NCHT
    ib, i4, pc = [ib0, ib1], [i40, i41], [pc0, pc1]
    rb, ob = [rb0, rb1], [ob0, ob1]
    si, sg, so = [si0, si1], [sg0, sg1], [so0, so1]

    def idx_copy(i, b):
        return pltpu.make_async_copy(
            idx_hbm.at[pl.ds((gbase + i) * _CK, _CK)], ib[b], si[b])

    def gather_copy(i, b):
        return pltpu.make_async_copy(table_hbm.at[i4[b]], rb[b], sg[b])

    def out_copy(i, b):
        g = gbase + i
        f = g // _CPF
        c0 = (g % _CPF) * _CK
        return pltpu.make_async_copy(
            ob[b], out_hbm.at[f, :, pl.ds(c0, _CK)], so[b])

    def prep(b):
        for j in range(_CK // _LANES):
            v = ib[b][pl.ds(j * _LANES, _LANES)]
            i4[b][pl.ds(j * _LANES, _LANES)] = lax.shift_right_logical(v, 2)
            pc[b][pl.ds(j * _LANES, _LANES)] = lax.shift_left(
                jnp.bitwise_and(v, 3), 5)

    def extract(b):
        def mbody(m, _):
            rowv = _iota16() + m * _LANES
            pcv = pc[b][pl.ds(m * _LANES, _LANES)]

            def tbody(t, _):
                wrapc = wrap_ref[t, :]
                for h in (0, 1):
                    dvec = wrapc + 16 * h if h else wrapc
                    colv = pcv + dvec
                    gv = plsc.load_gather(rb[b], [rowv, colv])
                    plsc.store_scatter(ob[b], [dvec, rowv], gv)
                return ()

            lax.fori_loop(0, _LANES, tbody, (), unroll=False)
            return ()

        lax.fori_loop(0, _CK // _LANES, mbody, (), unroll=False)

    idx_copy(0, 0).start()
    idx_copy(1, 1).start()

    def pair_body(k, _):
        for b in (0, 1):
            i = 2 * k + b
            pb = 1 - b
            idx_copy(i, b).wait()
            prep(b)
            gather_copy(i, b).start()

            def drain_prev():
                gather_copy(i - 1, pb).wait()

                @pl.when(i >= 3)
                def _():
                    out_copy(i - 3, pb).wait()

                extract(pb)
                out_copy(i - 1, pb).start()

            if b == 1:
                drain_prev()
            else:
                @pl.when(k > 0)
                def _():
                    drain_prev()

            @pl.when(i + 2 < _NCHT)
            def _():
                idx_copy(i + 2, b).start()
        return ()

    lax.fori_loop(0, _NCHT // 2, pair_body, (), unroll=False)

    last = _NCHT - 1
    gather_copy(last, 1).wait()
    out_copy(last - 2, 1).wait()
    extract(1)
    out_copy(last, 1).start()
    out_copy(last - 1, 0).wait()
    out_copy(last, 1).wait()


def kernel(indexes, embedding_weight):
    idx_flat = indexes.T.reshape(-1).astype(jnp.int32)
    tail = embedding_weight[_FULL_TILES * 128:]
    tbl_rm128 = _sc_transpose(embedding_weight.T, tail)
    out_t = _sc_gather_t(idx_flat, tbl_rm128)
    return out_t.transpose(2, 0, 1)


# speedup vs baseline: 2.0189x; 1.0057x over previous
import functools

import jax
import jax.numpy as jnp
from jax import lax
from jax.experimental import pallas as pl
from jax.experimental.pallas import tpu as pltpu
from jax.experimental.pallas import tpu_sc as plsc

NUM = 1000000
DIM = 32
BATCH = 16384
FIELDS = 26
B_TOTAL = BATCH * FIELDS

_info = plsc.get_sparse_core_info()
_NC, _NS = _info.num_cores, _info.num_subcores
_NW = _NC * _NS
_BPW = B_TOTAL // _NW
_CHUNK = 1664
_NCH = _BPW // _CHUNK

_LANES = 16
_JB = 6
_FULL_TILES = (NUM // 128)
_REM = NUM - _FULL_TILES * 128
_NGROUPS = _FULL_TILES // _JB

_mesh = plsc.VectorSubcoreMesh(core_axis_name="c", subcore_axis_name="s")


def _iota16(off=0):
    v = lax.broadcasted_iota(jnp.int32, (_LANES,), 0)
    return v + off if off else v


def _transpose_block(in_v, out_v, wrap_ref, outc_ref, nj):
    row0 = _iota16(0)
    row1 = _iota16(_LANES)

    def tbody(t, _):
        wrapc = wrap_ref[t, :]
        outc0 = outc_ref[t, :]
        for u in range(nj):
            for k in range(8):
                x0 = 128 * u + 16 * k
                colv = wrapc + x0
                outr = lax.shift_right_logical(colv, 2)
                g0 = plsc.load_gather(in_v, [row0, colv])
                plsc.store_scatter(out_v, [outr, outc0 + row0], g0)
                g1 = plsc.load_gather(in_v, [row1, colv])
                plsc.store_scatter(out_v, [outr, outc0 + row1], g1)
        return ()

    lax.fori_loop(0, _LANES, tbody, (), unroll=False)


@functools.partial(
    pl.kernel,
    mesh=_mesh,
    out_type=jax.ShapeDtypeStruct((NUM * DIM // 128, 128), jnp.float32),
    scratch_types=[
        pltpu.VMEM((DIM, 128 * _JB), jnp.float32),
        pltpu.VMEM((DIM, 128 * _JB), jnp.float32),
        pltpu.VMEM((32 * _JB, 128), jnp.float32),
        pltpu.VMEM((32 * _JB, 128), jnp.float32),
        pltpu.VMEM((_REM, DIM), jnp.float32),
        pltpu.VMEM((_LANES, _LANES), jnp.int32),
        pltpu.VMEM((_LANES, _LANES), jnp.int32),
        pltpu.SemaphoreType.DMA,
        pltpu.SemaphoreType.DMA,
        pltpu.SemaphoreType.DMA,
        pltpu.SemaphoreType.DMA,
    ],
    compiler_params=pltpu.CompilerParams(needs_layout_passes=False),
)
def _sc_transpose(table_t_hbm, tail_hbm, out_hbm, in0, in1, ob0, ob1, tv,
                  wrap_ref, outc_ref, si0, si1, so0, so1):
    for t in range(_LANES):
        w = jnp.bitwise_and(_iota16(t), _LANES - 1)
        wrap_ref[t, :] = w
        outc_ref[t, :] = lax.shift_left(jnp.bitwise_and(w, 3), 5)
    wid = lax.axis_index("s") * _NC + lax.axis_index("c")
    nbase = _NGROUPS // _NW
    extra = _NGROUPS - nbase * _NW
    cnt = jnp.where(wid < extra, nbase + 1, nbase)
    g0 = wid * nbase + jnp.minimum(wid, extra)

    ins, obs = [in0, in1], [ob0, ob1]
    sis, sos = [si0, si1], [so0, so1]

    def in_copy(g, b):
        return pltpu.make_async_copy(
            table_t_hbm.at[:, pl.ds(g * (128 * _JB), 128 * _JB)],
            ins[b], sis[b])

    def out_copy(g, b):
        return pltpu.make_async_copy(
            obs[b], out_hbm.at[pl.ds(g * (32 * _JB), 32 * _JB)], sos[b])

    in_copy(g0, 0).start()
    in_copy(g0 + 1, 1).start()

    def pair_body(k, _):
        for b in (0, 1):
            i = 2 * k + b
            g = g0 + i
            in_copy(g, b).wait()

            @pl.when(i >= 2)
            def _():
                out_copy(g - 2, b).wait()

            _transpose_block(ins[b], obs[b], wrap_ref, outc_ref, _JB)
            out_copy(g, b).start()

            @pl.when(i + 2 < cnt)
            def _():
                in_copy(g + 2, b).start()
        return ()

    lax.fori_loop(0, cnt // 2, pair_body, (), unroll=False)

    @pl.when(lax.rem(cnt, 2) == 1)
    def _():
        i = cnt - 1
        g = g0 + i
        in_copy(g, 0).wait()
        out_copy(g - 2, 0).wait()
        _transpose_block(ins[0], obs[0], wrap_ref, outc_ref, _JB)
        out_copy(g, 0).start()

    out_copy(g0, 0).wait()
    out_copy(g0, 1).wait()

    @pl.when(wid == _NW - 1)
    def _():
        pltpu.sync_copy(tail_hbm, tv)
        for r in range(_REM // 4):
            for q in range(4):
                ob0[r, pl.ds(32 * q, _LANES)] = \
                    tv[4 * r + q, pl.ds(0, _LANES)]
                ob0[r, pl.ds(32 * q + _LANES, _LANES)] = \
                    tv[4 * r + q, pl.ds(_LANES, _LANES)]
        pltpu.sync_copy(ob0.at[pl.ds(0, _REM // 4)],
                        out_hbm.at[pl.ds(_FULL_TILES * 32, _REM // 4)])


_CK = 256
_NCHT = _BPW // _CK
_CPF = BATCH // _CK


@functools.partial(
    pl.kernel,
    mesh=_mesh,
    out_type=jax.ShapeDtypeStruct((FIELDS, DIM, BATCH), jnp.float32),
    scratch_types=[
        pltpu.VMEM((_CK,), jnp.int32),
        pltpu.VMEM((_CK,), jnp.int32),
        pltpu.VMEM((_CK,), jnp.int32),
        pltpu.VMEM((_CK,), jnp.int32),
        pltpu.VMEM((_CK,), jnp.int32),
        pltpu.VMEM((_CK,), jnp.int32),
        pltpu.VMEM((_CK, 128), jnp.float32),
        pltpu.VMEM((_CK, 128), jnp.float32),
        pltpu.VMEM((DIM, _CK), jnp.float32),
        pltpu.VMEM((DIM, _CK), jnp.float32),
        pltpu.VMEM((_LANES, _LANES), jnp.int32),
        pltpu.SemaphoreType.DMA,
        pltpu.SemaphoreType.DMA,
        pltpu.SemaphoreType.DMA,
        pltpu.SemaphoreType.DMA,
        pltpu.SemaphoreType.DMA,
        pltpu.SemaphoreType.DMA,
    ],
    compiler_params=pltpu.CompilerParams(needs_layout_passes=False),
)
def _sc_gather_t(idx_hbm, table_hbm, out_hbm,
                 ib0, ib1, i40, i41, pc0, pc1, rb0, rb1, ob0, ob1,
                 wrap_ref, si0, si1, sg0, sg1, so0, so1):
    for t in range(_LANES):
        wrap_ref[t, :] = jnp.bitwise_and(_iota16(t), _LANES - 1)

    wid = lax.axis_index("s") * _NC + lax.axis_index("c")
    gbase = wid * _NCHT
    ib, i4, pc = [ib0, ib1], [i40, i41], [pc0, pc1]
    rb, ob = [rb0, rb1], [ob0, ob1]
    si, sg, so = [si0, si1], [sg0, sg1], [so0, so1]

    def idx_copy(i, b):
        return pltpu.make_async_copy(
            idx_hbm.at[pl.ds((gbase + i) * _CK, _CK)], ib[b], si[b])

    def gather_copy(i, b):
        return pltpu.make_async_copy(table_hbm.at[i4[b]], rb[b], sg[b])

    def out_copy(i, b):
        g = gbase + i
        f = g // _CPF
        c0 = (g % _CPF) * _CK
        return pltpu.make_async_copy(
            ob[b], out_hbm.at[f, :, pl.ds(c0, _CK)], so[b])

    def prep(b):
        for j in range(_CK // _LANES):
            v = ib[b][pl.ds(j * _LANES, _LANES)]
            i4[b][pl.ds(j * _LANES, _LANES)] = lax.shift_right_logical(v, 2)
            pc[b][pl.ds(j * _LANES, _LANES)] = lax.shift_left(
                jnp.bitwise_and(v, 3), 5)

    def extract(b):
        def mbody(m, _):
            rowv = _iota16() + m * _LANES
            pcv = pc[b][pl.ds(m * _LANES, _LANES)]

            def tbody(t, _):
                wrapc = wrap_ref[t, :]
                for h in (0, 1):
                    dvec = wrapc + 16 * h if h else wrapc
                    colv = pcv + dvec
                    gv = plsc.load_gather(rb[b], [rowv, colv])
                    plsc.store_scatter(ob[b], [dvec, rowv], gv)
                return ()

            lax.fori_loop(0, _LANES, tbody, (), unroll=False)
            return ()

        lax.fori_loop(0, _CK // _LANES, mbody, (), unroll=False)

    idx_copy(0, 0).start()
    idx_copy(1, 1).start()

    def pair_body(k, _):
        for b in (0, 1):
            i = 2 * k + b
            pb = 1 - b
            idx_copy(i, b).wait()
            prep(b)
            gather_copy(i, b).start()

            def drain_prev():
                gather_copy(i - 1, pb).wait()

                @pl.when(i >= 3)
                def _():
                    out_copy(i - 3, pb).wait()

                extract(pb)
                out_copy(i - 1, pb).start()

            if b == 1:
                drain_prev()
            else:
                @pl.when(k > 0)
                def _():
                    drain_prev()

            @pl.when(i + 2 < _NCHT)
            def _():
                idx_copy(i + 2, b).start()
        return ()

    lax.fori_loop(0, _NCHT // 2, pair_body, (), unroll=False)

    last = _NCHT - 1
    gather_copy(last, 1).wait()
    out_copy(last - 2, 1).wait()
    extract(1)
    out_copy(last, 1).start()
    out_copy(last - 1, 0).wait()
    out_copy(last, 1).wait()


def kernel(indexes, embedding_weight):
    idx_flat = indexes.T.reshape(-1).astype(jnp.int32)
    tail = embedding_weight[_FULL_TILES * 128:]
    tbl_rm128 = _sc_transpose(embedding_weight.T, tail)
    out_t = _sc_gather_t(idx_flat, tbl_rm128)
    return out_t.transpose(2, 0, 1)
